# baseline (speedup 1.0000x reference)
import functools

import jax
import jax.numpy as jnp
from jax import lax
from jax.experimental import pallas as pl
from jax.experimental.pallas import tpu as pltpu
from jax.experimental.pallas import tpu_sc as plsc

IN_CHANNELS = 256
HIDDEN_DIM = 128
NUM_EXPERTS = 4
B = 32
H = 64
W = 64
TC_B = 24
SC_B = B - TC_B
QUART = H // 4
CHUNK = 2
NCHUNK = QUART // CHUNK
BB = 2

NC = 2
NS = 16
LANES = 16


def _mlp_rows(pooled, w1, b1, w2, b2):
    h = jnp.dot(pooled, w1, preferred_element_type=jnp.float32) + b1
    h = 0.5 * h * (1.0 + lax.erf(h * 0.7071067811865476))
    logits = jnp.dot(h, w2, preferred_element_type=jnp.float32) + b2
    m = jnp.max(logits, axis=-1, keepdims=True)
    e = jnp.exp(logits - m)
    return e / jnp.sum(e, axis=-1, keepdims=True)


def _tc_main_kernel(x_ref, w1_ref, b1_ref, w2_ref, b2_ref, out_ref):
    g = pl.program_id(0)
    hw = x_ref.shape[1] * x_ref.shape[2]
    rows = []
    for i in range(BB):
        x = x_ref[i]
        part = jnp.sum(x, axis=0)
        rows.append(jnp.sum(part, axis=0))
    pooled = jnp.stack(rows, axis=0) * (1.0 / hw)
    out_ref[pl.ds(g * BB, BB), :] = _mlp_rows(
        pooled, w1_ref[...], b1_ref[...], w2_ref[...], b2_ref[...])


def _tc_tail_kernel(p_ref, w1_ref, b1_ref, w2_ref, b2_ref, out_ref):
    p = p_ref[...]
    pooled = jnp.sum(p, axis=0) * (1.0 / (H * W))
    out_ref[...] = _mlp_rows(
        pooled, w1_ref[...], b1_ref[...], w2_ref[...], b2_ref[...])


def _sc_pool_kernel(x_hbm, out_hbm, buf, acc, sem0, sem1):
    c = lax.axis_index("c")
    s = lax.axis_index("s")
    wid = s * NC + c
    sample = TC_B + wid // 4
    q = wid % 4
    h0 = q * QUART

    zeros = jnp.zeros((LANES,), jnp.float32)

    def zbody(j, _):
        for k in range(IN_CHANNELS // LANES):
            acc[j, pl.ds(k * LANES, LANES)] = zeros
        return 0

    lax.fori_loop(0, W, zbody, 0)

    sems = (sem0, sem1)
    copies = [None, None]
    copies[0] = pltpu.async_copy(
        x_hbm.at[sample, pl.ds(h0, CHUNK)], buf.at[0], sems[0])
    for t in range(NCHUNK):
        cur = t % 2
        nxt = (t + 1) % 2
        copies[cur].wait()
        if t + 1 < NCHUNK:
            copies[nxt] = pltpu.async_copy(
                x_hbm.at[sample, pl.ds(h0 + (t + 1) * CHUNK, CHUNK)],
                buf.at[nxt], sems[nxt])

        def abody(j, _, cur=cur):
            for h in range(CHUNK):
                for k in range(IN_CHANNELS // LANES):
                    sl = pl.ds(k * LANES, LANES)
                    plsc.addupdate(acc.at[j, sl], buf[cur, h, j, sl])
            return 0

        lax.fori_loop(0, W, abody, 0)

    def rbody(j, _):
        for k in range(IN_CHANNELS // LANES):
            sl = pl.ds(k * LANES, LANES)
            plsc.addupdate(acc.at[0, sl], acc[j, sl])
        return 0

    lax.fori_loop(1, W, rbody, 0)

    pltpu.sync_copy(acc.at[0], out_hbm.at[q, sample - TC_B])


@jax.jit
def kernel(img_emb, W1, b1, W2, b2):
    x = img_emb.transpose(0, 2, 3, 1)
    b1r = b1.reshape(1, HIDDEN_DIM)
    b2r = b2.reshape(1, NUM_EXPERTS)

    sc_partials = pl.kernel(
        _sc_pool_kernel,
        out_type=jax.ShapeDtypeStruct((4, SC_B, IN_CHANNELS), jnp.float32),
        mesh=plsc.VectorSubcoreMesh(core_axis_name="c", subcore_axis_name="s"),
        scratch_types=[
            pltpu.VMEM((2, CHUNK, W, IN_CHANNELS), jnp.float32),
            pltpu.VMEM((W, IN_CHANNELS), jnp.float32),
            pltpu.SemaphoreType.DMA,
            pltpu.SemaphoreType.DMA,
        ],
    )(x)

    out_main = pl.pallas_call(
        _tc_main_kernel,
        grid=(TC_B // BB,),
        in_specs=[
            pl.BlockSpec((BB, H, W, IN_CHANNELS), lambda g: (g, 0, 0, 0)),
            pl.BlockSpec((IN_CHANNELS, HIDDEN_DIM), lambda g: (0, 0)),
            pl.BlockSpec((1, HIDDEN_DIM), lambda g: (0, 0)),
            pl.BlockSpec((HIDDEN_DIM, NUM_EXPERTS), lambda g: (0, 0)),
            pl.BlockSpec((1, NUM_EXPERTS), lambda g: (0, 0)),
        ],
        out_specs=pl.BlockSpec((TC_B, NUM_EXPERTS), lambda g: (0, 0)),
        out_shape=jax.ShapeDtypeStruct((TC_B, NUM_EXPERTS), jnp.float32),
    )(x, W1, b1r, W2, b2r)

    out_tail = pl.pallas_call(
        _tc_tail_kernel,
        in_specs=[
            pl.BlockSpec((4, SC_B, IN_CHANNELS), lambda: (0, 0, 0)),
            pl.BlockSpec((IN_CHANNELS, HIDDEN_DIM), lambda: (0, 0)),
            pl.BlockSpec((1, HIDDEN_DIM), lambda: (0, 0)),
            pl.BlockSpec((HIDDEN_DIM, NUM_EXPERTS), lambda: (0, 0)),
            pl.BlockSpec((1, NUM_EXPERTS), lambda: (0, 0)),
        ],
        out_specs=pl.BlockSpec((SC_B, NUM_EXPERTS), lambda: (0, 0)),
        out_shape=jax.ShapeDtypeStruct((SC_B, NUM_EXPERTS), jnp.float32),
    )(sc_partials, W1, b1r, W2, b2r)

    return jnp.concatenate([out_main, out_tail], axis=0)


# speedup vs baseline: 1.4803x; 1.4803x over previous
import functools

import jax
import jax.numpy as jnp
from jax import lax
from jax.experimental import pallas as pl
from jax.experimental.pallas import tpu as pltpu
from jax.experimental.pallas import tpu_sc as plsc

IN_CHANNELS = 256
HIDDEN_DIM = 128
NUM_EXPERTS = 4
B = 32
H = 64
W = 64
TC_B = 28
SC_B = B - TC_B
WPS = 8
QUART = H // WPS
CHUNK = 2
NCHUNK = QUART // CHUNK
BB = 2

NC = 2
NS = 16
LANES = 16


def _mlp_rows(pooled, w1, b1, w2, b2):
    h = jnp.dot(pooled, w1, preferred_element_type=jnp.float32) + b1
    h = 0.5 * h * (1.0 + lax.erf(h * 0.7071067811865476))
    logits = jnp.dot(h, w2, preferred_element_type=jnp.float32) + b2
    m = jnp.max(logits, axis=-1, keepdims=True)
    e = jnp.exp(logits - m)
    return e / jnp.sum(e, axis=-1, keepdims=True)


def _tc_main_kernel(x_ref, w1_ref, b1_ref, w2_ref, b2_ref, out_ref):
    g = pl.program_id(0)
    hw = x_ref.shape[1] * x_ref.shape[2]
    rows = []
    for i in range(BB):
        x = x_ref[i]
        part = jnp.sum(x, axis=0)
        rows.append(jnp.sum(part, axis=0))
    pooled = jnp.stack(rows, axis=0) * (1.0 / hw)
    out_ref[pl.ds(g * BB, BB), :] = _mlp_rows(
        pooled, w1_ref[...], b1_ref[...], w2_ref[...], b2_ref[...])


def _tc_tail_kernel(p_ref, w1_ref, b1_ref, w2_ref, b2_ref, out_ref):
    p = p_ref[...]
    pooled = jnp.sum(p, axis=0) * (1.0 / (H * W))
    out_ref[...] = _mlp_rows(
        pooled, w1_ref[...], b1_ref[...], w2_ref[...], b2_ref[...])


def _sc_pool_kernel(x_hbm, out_hbm, buf, acc, sem0, sem1):
    c = lax.axis_index("c")
    s = lax.axis_index("s")
    wid = s * NC + c
    sample = TC_B + wid // WPS
    q = wid % WPS
    h0 = q * QUART

    zeros = jnp.zeros((LANES,), jnp.float32)

    def zbody(j, _):
        for k in range(IN_CHANNELS // LANES):
            acc[j, pl.ds(k * LANES, LANES)] = zeros
        return 0

    lax.fori_loop(0, W, zbody, 0)

    sems = (sem0, sem1)
    copies = [None, None]
    copies[0] = pltpu.async_copy(
        x_hbm.at[sample, pl.ds(h0, CHUNK)], buf.at[0], sems[0])
    for t in range(NCHUNK):
        cur = t % 2
        nxt = (t + 1) % 2
        copies[cur].wait()
        if t + 1 < NCHUNK:
            copies[nxt] = pltpu.async_copy(
                x_hbm.at[sample, pl.ds(h0 + (t + 1) * CHUNK, CHUNK)],
                buf.at[nxt], sems[nxt])

        def abody(j, _, cur=cur):
            for h in range(CHUNK):
                for k in range(IN_CHANNELS // LANES):
                    sl = pl.ds(k * LANES, LANES)
                    plsc.addupdate(acc.at[j, sl], buf[cur, h, j, sl])
            return 0

        lax.fori_loop(0, W, abody, 0)

    def rbody(j, _):
        for k in range(IN_CHANNELS // LANES):
            sl = pl.ds(k * LANES, LANES)
            plsc.addupdate(acc.at[0, sl], acc[j, sl])
        return 0

    lax.fori_loop(1, W, rbody, 0)

    pltpu.sync_copy(acc.at[0], out_hbm.at[q, sample - TC_B])


@jax.jit
def kernel(img_emb, W1, b1, W2, b2):
    x = img_emb.transpose(0, 2, 3, 1)
    b1r = b1.reshape(1, HIDDEN_DIM)
    b2r = b2.reshape(1, NUM_EXPERTS)

    sc_partials = pl.kernel(
        _sc_pool_kernel,
        out_type=jax.ShapeDtypeStruct((WPS, SC_B, IN_CHANNELS), jnp.float32),
        mesh=plsc.VectorSubcoreMesh(core_axis_name="c", subcore_axis_name="s"),
        scratch_types=[
            pltpu.VMEM((2, CHUNK, W, IN_CHANNELS), jnp.float32),
            pltpu.VMEM((W, IN_CHANNELS), jnp.float32),
            pltpu.SemaphoreType.DMA,
            pltpu.SemaphoreType.DMA,
        ],
    )(x)

    out_main = pl.pallas_call(
        _tc_main_kernel,
        grid=(TC_B // BB,),
        in_specs=[
            pl.BlockSpec((BB, H, W, IN_CHANNELS), lambda g: (g, 0, 0, 0)),
            pl.BlockSpec((IN_CHANNELS, HIDDEN_DIM), lambda g: (0, 0)),
            pl.BlockSpec((1, HIDDEN_DIM), lambda g: (0, 0)),
            pl.BlockSpec((HIDDEN_DIM, NUM_EXPERTS), lambda g: (0, 0)),
            pl.BlockSpec((1, NUM_EXPERTS), lambda g: (0, 0)),
        ],
        out_specs=pl.BlockSpec((TC_B, NUM_EXPERTS), lambda g: (0, 0)),
        out_shape=jax.ShapeDtypeStruct((TC_B, NUM_EXPERTS), jnp.float32),
    )(x, W1, b1r, W2, b2r)

    out_tail = pl.pallas_call(
        _tc_tail_kernel,
        in_specs=[
            pl.BlockSpec((WPS, SC_B, IN_CHANNELS), lambda: (0, 0, 0)),
            pl.BlockSpec((IN_CHANNELS, HIDDEN_DIM), lambda: (0, 0)),
            pl.BlockSpec((1, HIDDEN_DIM), lambda: (0, 0)),
            pl.BlockSpec((HIDDEN_DIM, NUM_EXPERTS), lambda: (0, 0)),
            pl.BlockSpec((1, NUM_EXPERTS), lambda: (0, 0)),
        ],
        out_specs=pl.BlockSpec((SC_B, NUM_EXPERTS), lambda: (0, 0)),
        out_shape=jax.ShapeDtypeStruct((SC_B, NUM_EXPERTS), jnp.float32),
    )(sc_partials, W1, b1r, W2, b2r)

    return jnp.concatenate([out_main, out_tail], axis=0)


# speedup vs baseline: 1.4872x; 1.0047x over previous
import functools

import jax
import jax.numpy as jnp
from jax import lax
from jax.experimental import pallas as pl
from jax.experimental.pallas import tpu as pltpu
from jax.experimental.pallas import tpu_sc as plsc

IN_CHANNELS = 256
HIDDEN_DIM = 128
NUM_EXPERTS = 4
B = 32
H = 64
W = 64
TC_B = 30
SC_B = B - TC_B
WPS = 16
QUART = H // WPS
CHUNK = 2
NCHUNK = QUART // CHUNK
BB = 2

NC = 2
NS = 16
LANES = 16


def _mlp_rows(pooled, w1, b1, w2, b2):
    h = jnp.dot(pooled, w1, preferred_element_type=jnp.float32) + b1
    h = 0.5 * h * (1.0 + lax.erf(h * 0.7071067811865476))
    logits = jnp.dot(h, w2, preferred_element_type=jnp.float32) + b2
    m = jnp.max(logits, axis=-1, keepdims=True)
    e = jnp.exp(logits - m)
    return e / jnp.sum(e, axis=-1, keepdims=True)


def _tc_main_kernel(x_ref, w1_ref, b1_ref, w2_ref, b2_ref, out_ref):
    g = pl.program_id(0)
    hw = x_ref.shape[1] * x_ref.shape[2]
    rows = []
    for i in range(BB):
        x = x_ref[i]
        part = jnp.sum(x, axis=0)
        rows.append(jnp.sum(part, axis=0))
    pooled = jnp.stack(rows, axis=0) * (1.0 / hw)
    out_ref[pl.ds(g * BB, BB), :] = _mlp_rows(
        pooled, w1_ref[...], b1_ref[...], w2_ref[...], b2_ref[...])


def _tc_tail_kernel(p_ref, w1_ref, b1_ref, w2_ref, b2_ref, out_ref):
    p = p_ref[...]
    pooled = jnp.sum(p, axis=0) * (1.0 / (H * W))
    out_ref[...] = _mlp_rows(
        pooled, w1_ref[...], b1_ref[...], w2_ref[...], b2_ref[...])


def _sc_pool_kernel(x_hbm, out_hbm, buf, acc, sem0, sem1):
    c = lax.axis_index("c")
    s = lax.axis_index("s")
    wid = s * NC + c
    sample = TC_B + wid // WPS
    q = wid % WPS
    h0 = q * QUART

    zeros = jnp.zeros((LANES,), jnp.float32)

    def zbody(j, _):
        for k in range(IN_CHANNELS // LANES):
            acc[j, pl.ds(k * LANES, LANES)] = zeros
        return 0

    lax.fori_loop(0, W, zbody, 0)

    sems = (sem0, sem1)
    copies = [None, None]
    copies[0] = pltpu.async_copy(
        x_hbm.at[sample, pl.ds(h0, CHUNK)], buf.at[0], sems[0])
    for t in range(NCHUNK):
        cur = t % 2
        nxt = (t + 1) % 2
        copies[cur].wait()
        if t + 1 < NCHUNK:
            copies[nxt] = pltpu.async_copy(
                x_hbm.at[sample, pl.ds(h0 + (t + 1) * CHUNK, CHUNK)],
                buf.at[nxt], sems[nxt])

        def abody(j, _, cur=cur):
            for h in range(CHUNK):
                for k in range(IN_CHANNELS // LANES):
                    sl = pl.ds(k * LANES, LANES)
                    plsc.addupdate(acc.at[j, sl], buf[cur, h, j, sl])
            return 0

        lax.fori_loop(0, W, abody, 0)

    def rbody(j, _):
        for k in range(IN_CHANNELS // LANES):
            sl = pl.ds(k * LANES, LANES)
            plsc.addupdate(acc.at[0, sl], acc[j, sl])
        return 0

    lax.fori_loop(1, W, rbody, 0)

    pltpu.sync_copy(acc.at[0], out_hbm.at[q, sample - TC_B])


@jax.jit
def kernel(img_emb, W1, b1, W2, b2):
    x = img_emb.transpose(0, 2, 3, 1)
    b1r = b1.reshape(1, HIDDEN_DIM)
    b2r = b2.reshape(1, NUM_EXPERTS)

    sc_partials = pl.kernel(
        _sc_pool_kernel,
        out_type=jax.ShapeDtypeStruct((WPS, SC_B, IN_CHANNELS), jnp.float32),
        mesh=plsc.VectorSubcoreMesh(core_axis_name="c", subcore_axis_name="s"),
        scratch_types=[
            pltpu.VMEM((2, CHUNK, W, IN_CHANNELS), jnp.float32),
            pltpu.VMEM((W, IN_CHANNELS), jnp.float32),
            pltpu.SemaphoreType.DMA,
            pltpu.SemaphoreType.DMA,
        ],
    )(x)

    out_main = pl.pallas_call(
        _tc_main_kernel,
        grid=(TC_B // BB,),
        in_specs=[
            pl.BlockSpec((BB, H, W, IN_CHANNELS), lambda g: (g, 0, 0, 0)),
            pl.BlockSpec((IN_CHANNELS, HIDDEN_DIM), lambda g: (0, 0)),
            pl.BlockSpec((1, HIDDEN_DIM), lambda g: (0, 0)),
            pl.BlockSpec((HIDDEN_DIM, NUM_EXPERTS), lambda g: (0, 0)),
            pl.BlockSpec((1, NUM_EXPERTS), lambda g: (0, 0)),
        ],
        out_specs=pl.BlockSpec((TC_B, NUM_EXPERTS), lambda g: (0, 0)),
        out_shape=jax.ShapeDtypeStruct((TC_B, NUM_EXPERTS), jnp.float32),
    )(x, W1, b1r, W2, b2r)

    out_tail = pl.pallas_call(
        _tc_tail_kernel,
        in_specs=[
            pl.BlockSpec((WPS, SC_B, IN_CHANNELS), lambda: (0, 0, 0)),
            pl.BlockSpec((IN_CHANNELS, HIDDEN_DIM), lambda: (0, 0)),
            pl.BlockSpec((1, HIDDEN_DIM), lambda: (0, 0)),
            pl.BlockSpec((HIDDEN_DIM, NUM_EXPERTS), lambda: (0, 0)),
            pl.BlockSpec((1, NUM_EXPERTS), lambda: (0, 0)),
        ],
        out_specs=pl.BlockSpec((SC_B, NUM_EXPERTS), lambda: (0, 0)),
        out_shape=jax.ShapeDtypeStruct((SC_B, NUM_EXPERTS), jnp.float32),
    )(sc_partials, W1, b1r, W2, b2r)

    return jnp.concatenate([out_main, out_tail], axis=0)


# speedup vs baseline: 2.1092x; 1.4182x over previous
import jax
import jax.numpy as jnp
from jax.experimental import pallas as pl

IN_CHANNELS = 256
HIDDEN_DIM = 128
NUM_EXPERTS = 4
BB = 2


def _gate_kernel(x_ref, w1_ref, b1_ref, w2_ref, b2_ref, out_ref):
    g = pl.program_id(0)
    hw = x_ref.shape[1] * x_ref.shape[2]
    rows = []
    for i in range(BB):
        x = x_ref[i]
        part = jnp.sum(x, axis=0)
        rows.append(jnp.sum(part, axis=0))
    pooled = jnp.stack(rows, axis=0) * (1.0 / hw)
    h = jnp.dot(pooled, w1_ref[...], preferred_element_type=jnp.float32)
    h = h + b1_ref[...]
    h = 0.5 * h * (1.0 + jax.lax.erf(h * 0.7071067811865476))
    logits = jnp.dot(h, w2_ref[...], preferred_element_type=jnp.float32)
    logits = logits + b2_ref[...]
    m = jnp.max(logits, axis=-1, keepdims=True)
    e = jnp.exp(logits - m)
    weights = e / jnp.sum(e, axis=-1, keepdims=True)
    out_ref[pl.ds(g * BB, BB), :] = weights


@jax.jit
def kernel(img_emb, W1, b1, W2, b2):
    B, C, H, W = img_emb.shape
    x = img_emb.transpose(0, 2, 3, 1)
    b1r = b1.reshape(1, HIDDEN_DIM)
    b2r = b2.reshape(1, NUM_EXPERTS)
    out = pl.pallas_call(
        _gate_kernel,
        grid=(B // BB,),
        in_specs=[
            pl.BlockSpec((BB, H, W, C), lambda g: (g, 0, 0, 0)),
            pl.BlockSpec((C, HIDDEN_DIM), lambda g: (0, 0)),
            pl.BlockSpec((1, HIDDEN_DIM), lambda g: (0, 0)),
            pl.BlockSpec((HIDDEN_DIM, NUM_EXPERTS), lambda g: (0, 0)),
            pl.BlockSpec((1, NUM_EXPERTS), lambda g: (0, 0)),
        ],
        out_specs=pl.BlockSpec((B, NUM_EXPERTS), lambda g: (0, 0)),
        out_shape=jax.ShapeDtypeStruct((B, NUM_EXPERTS), jnp.float32),
    )(x, W1, b1r, W2, b2r)
    return out
